# Initial kernel scaffold; baseline (speedup 1.0000x reference)
import jax
import jax.numpy as jnp
from jax.experimental import pallas as pl


def kernel(x, edge_index, edge_type, weight, root, bias):
    raise NotImplementedError("write your pallas kernel here")


# speedup vs baseline: 3.2801x; 3.2801x over previous
import functools

import jax
import jax.numpy as jnp
from jax.experimental import pallas as pl

_BN = 1000
_BD = 256


def _matmul_block(x_ref, w_ref, o_ref):
    o_ref[...] = jnp.dot(x_ref[...], w_ref[...],
                         preferred_element_type=jnp.float32)


def _dense_transform(x, w_all):
    n, d = x.shape
    kout = w_all.shape[1]
    grid = (n // _BN, kout // _BD)
    return pl.pallas_call(
        _matmul_block,
        grid=grid,
        in_specs=[
            pl.BlockSpec((_BN, d), lambda i, j: (i, 0)),
            pl.BlockSpec((d, _BD), lambda i, j: (0, j)),
        ],
        out_specs=pl.BlockSpec((_BN, _BD), lambda i, j: (i, j)),
        out_shape=jax.ShapeDtypeStruct((n, kout), jnp.float32),
    )(x, w_all)


def kernel(x, edge_index, edge_type, weight, root, bias):
    n, d = x.shape
    r = weight.shape[0]
    src = edge_index[0]
    dst = edge_index[1]

    w_all = jnp.concatenate(
        [weight.transpose(1, 0, 2).reshape(d, r * d), root], axis=1)
    z = _dense_transform(x, w_all)
    zr = z[:, : r * d].reshape(n, r, d)
    zroot = z[:, r * d:]

    cnt = jnp.zeros((n, r), jnp.float32).at[dst, edge_type].add(1.0)
    scale = 1.0 / jnp.clip(cnt, 1.0)
    msg = zr[src, edge_type] * scale[dst, edge_type][:, None]
    agg = jnp.zeros((n, d), jnp.float32).at[dst].add(msg)

    out = agg + zroot + bias
    return (out, edge_index, edge_type)
